# baseline (speedup 1.0000x reference)
import jax
import jax.numpy as jnp
from jax.experimental import pallas as pl
from jax.experimental.pallas import tpu as pltpu

B = 8
C = 768
HW = 4096
UNIT = 2048
N_UNITS = B * (HW // UNIT)
N_CH = 4
CH = UNIT // N_CH


def _unit_src(t):
    per_b = HW // UNIT
    return t // per_b, (t % per_b) * UNIT


def _matmul_kernel(w_ref, x_ref, o_ref, inbuf, outbuf, insem, outsem):
    def in_copy(t):
        b, off = _unit_src(t)
        slot = t % 2
        return [
            pltpu.make_async_copy(
                x_ref.at[b, :, pl.ds(off + k * CH, CH)],
                inbuf.at[slot, :, pl.ds(k * CH, CH)],
                insem.at[slot, k],
            )
            for k in range(N_CH)
        ]

    def out_copy(t):
        b, off = _unit_src(t)
        slot = t % 2
        return [
            pltpu.make_async_copy(
                outbuf.at[slot, :, pl.ds(k * CH, CH)],
                o_ref.at[b, :, pl.ds(off + k * CH, CH)],
                outsem.at[slot, k],
            )
            for k in range(N_CH)
        ]

    for cp in in_copy(0):
        cp.start()
    for t in range(N_UNITS):
        if t + 1 < N_UNITS:
            for cp in in_copy(t + 1):
                cp.start()
        for cp in in_copy(t):
            cp.wait()
        if t >= 2:
            for cp in out_copy(t - 2):
                cp.wait()
        slot = t % 2
        res = jnp.dot(
            w_ref[...],
            inbuf[slot].astype(jnp.bfloat16),
            preferred_element_type=jnp.float32,
        )
        outbuf[slot] = res.astype(jnp.bfloat16)
        for cp in out_copy(t):
            cp.start()
    for cp in out_copy(N_UNITS - 2):
        cp.wait()
    for cp in out_copy(N_UNITS - 1):
        cp.wait()


def kernel(inputs, W):
    b, c, h, w = inputs.shape
    filters = W.shape[0]
    hw = h * w
    x = inputs.reshape(b, c, hw)
    w_bf16 = W.astype(jnp.bfloat16)

    out = pl.pallas_call(
        _matmul_kernel,
        in_specs=[
            pl.BlockSpec(memory_space=pltpu.VMEM),
            pl.BlockSpec(memory_space=pl.ANY),
        ],
        out_specs=pl.BlockSpec(memory_space=pl.ANY),
        out_shape=jax.ShapeDtypeStruct((b, filters, hw), jnp.bfloat16),
        scratch_shapes=[
            pltpu.VMEM((2, C, UNIT), jnp.float32),
            pltpu.VMEM((2, C, UNIT), jnp.bfloat16),
            pltpu.SemaphoreType.DMA((2, N_CH)),
            pltpu.SemaphoreType.DMA((2, N_CH)),
        ],
    )(w_bf16, x)
    return out.reshape(b, filters, h, w).astype(jnp.float32)
